# baseline (speedup 1.0000x reference)
import functools

import jax
import jax.numpy as jnp
from jax.experimental import pallas as pl
from jax.experimental.pallas import tpu as pltpu

E = 8
TOP_K = 2
D = 1024
FF = 1024
T = 2048

TT = 8
TB = T // TT


def _moe_body(x_ref, rnd_ref, wg_ref, wu_ref, wd_ref,
              swg_ref, swu_ref, swd_ref, out_ref):
    i = pl.program_id(0)
    tt = pl.program_id(1)

    xt = x_ref[...]
    wg = jnp.where(i == E, swg_ref[...], wg_ref[0])
    wu = jnp.where(i == E, swu_ref[...], wu_ref[0])
    wd = jnp.where(i == E, swd_ref[...], wd_ref[0])
    a = jnp.dot(xt, wg, preferred_element_type=jnp.float32)
    b = jnp.dot(xt, wu, preferred_element_type=jnp.float32)
    h = a * jax.lax.logistic(a) * b
    o = jnp.dot(h, wd, preferred_element_type=jnp.float32)

    r = rnd_ref[...]
    m = jnp.max(r, axis=1, keepdims=True)
    ex = jnp.exp(r - m)
    s = ex / jnp.sum(ex, axis=1, keepdims=True)
    m1 = jnp.max(s, axis=1, keepdims=True)
    is1 = s == m1
    m2 = jnp.max(jnp.where(is1, -jnp.inf, s), axis=1, keepdims=True)
    is2 = jnp.logical_and(s == m2, jnp.logical_not(is1))
    wt = s * jnp.logical_or(is1, is2).astype(s.dtype)
    lane = jax.lax.broadcasted_iota(jnp.int32, (TB, E), 1)
    col = jnp.sum(jnp.where(lane == i, wt, 0.0), axis=1, keepdims=True)
    col = jnp.where(i == E, 1.0, col)

    contrib = o * col
    rows = pl.ds(tt * TB, TB)

    @pl.when(i == 0)
    def _():
        out_ref[rows, :] = contrib

    @pl.when(i > 0)
    def _():
        out_ref[rows, :] += contrib


@jax.jit
def kernel(x, gate_w, w_gate, w_up, w_down, sw_gate, sw_up, sw_down):
    del gate_w
    rnd = jax.random.normal(jax.random.key(42), (T, E), dtype=jnp.float32)

    grid = (E + 1, TT)
    out = pl.pallas_call(
        _moe_body,
        grid=grid,
        in_specs=[
            pl.BlockSpec((TB, D), lambda i, tt: (tt, 0)),
            pl.BlockSpec((TB, E), lambda i, tt: (tt, 0)),
            pl.BlockSpec((1, D, FF), lambda i, tt: (jnp.minimum(i, E - 1), 0, 0)),
            pl.BlockSpec((1, D, FF), lambda i, tt: (jnp.minimum(i, E - 1), 0, 0)),
            pl.BlockSpec((1, FF, D), lambda i, tt: (jnp.minimum(i, E - 1), 0, 0)),
            pl.BlockSpec((D, FF), lambda i, tt: (0, 0)),
            pl.BlockSpec((D, FF), lambda i, tt: (0, 0)),
            pl.BlockSpec((FF, D), lambda i, tt: (0, 0)),
        ],
        out_specs=pl.BlockSpec((T, D), lambda i, tt: (0, 0)),
        out_shape=jax.ShapeDtypeStruct((T, D), jnp.float32),
    )(x, rnd, w_gate, w_up, w_down, sw_gate, sw_up, sw_down)
    return out


# speedup vs baseline: 1.3988x; 1.3988x over previous
import functools

import jax
import jax.numpy as jnp
import numpy as np
from jax import lax
from jax.experimental import pallas as pl
from jax.experimental.pallas import tpu as pltpu
from jax.experimental.pallas import tpu_sc as plsc

E = 8
TOP_K = 2
D = 1024
FF = 1024
T = 2048

M = 576
G = E * M
NW = 32


def _routing_tables():
    rnd = np.asarray(jax.random.normal(jax.random.key(42), (T, E),
                                       dtype=jnp.float32))
    s64 = np.exp(rnd.astype(np.float64))
    scores = (s64 / s64.sum(axis=1, keepdims=True)).astype(np.float32)
    order = np.argsort(-rnd, axis=1, kind="stable")
    top1, top2 = order[:, 0], order[:, 1]
    s1 = scores[np.arange(T), top1]
    s2 = scores[np.arange(T), top2]

    gidx = np.zeros((G,), np.int32)
    gscale = np.zeros((G,), np.float32)
    inv = np.zeros((TOP_K * T,), np.int32)
    for e in range(E):
        m0 = top1 == e
        m1 = top2 == e
        toks = np.where(m0 | m1)[0]
        n = len(toks)
        assert n <= M
        base = e * M
        gidx[base:base + n] = toks
        gscale[base:base + n] = np.where(m0[toks], s1[toks], s2[toks])
        rank = np.where(m0[toks], 0, 1)
        inv[rank * T + toks] = base + np.arange(n, dtype=np.int64)
    return gidx, gscale, inv


_GIDX, _GSCALE, _INV = _routing_tables()


@functools.cache
def _make_sc_gather(n_src, n_rows, chunk):
    assert n_rows % NW == 0
    rpw = n_rows // NW
    assert rpw % chunk == 0
    nch = rpw // chunk
    mesh = plsc.VectorSubcoreMesh(core_axis_name="c", subcore_axis_name="s")

    @functools.partial(
        pl.kernel,
        out_type=jax.ShapeDtypeStruct((n_rows, D), jnp.float32),
        mesh=mesh,
        scratch_types=[
            pltpu.VMEM((nch, chunk), jnp.int32),
            pltpu.VMEM((chunk, D), jnp.float32),
            pltpu.SemaphoreType.DMA,
        ],
    )
    def gather(src_hbm, idx_hbm, out_hbm, idx_v, buf, sem):
        wid = lax.axis_index("s") * 2 + lax.axis_index("c")
        pltpu.sync_copy(idx_hbm.at[wid], idx_v)
        for c in range(nch):
            pltpu.async_copy(src_hbm.at[idx_v.at[c]], buf, sem).wait()
            pltpu.sync_copy(
                buf, out_hbm.at[pl.ds(wid * rpw + c * chunk, chunk)])

    return gather


def _expert_body(xg_ref, wg_ref, wu_ref, wd_ref, gs_ref, yg_ref):
    xt = xg_ref[...]
    a = jnp.dot(xt, wg_ref[0], preferred_element_type=jnp.float32)
    b = jnp.dot(xt, wu_ref[0], preferred_element_type=jnp.float32)
    h = a * lax.logistic(a) * b
    o = jnp.dot(h, wd_ref[0], preferred_element_type=jnp.float32)
    yg_ref[...] = o * gs_ref[...]


def _experts(xg, w_gate, w_up, w_down, gscale):
    return pl.pallas_call(
        _expert_body,
        grid=(E,),
        in_specs=[
            pl.BlockSpec((M, D), lambda e: (e, 0)),
            pl.BlockSpec((1, D, FF), lambda e: (e, 0, 0)),
            pl.BlockSpec((1, D, FF), lambda e: (e, 0, 0)),
            pl.BlockSpec((1, FF, D), lambda e: (e, 0, 0)),
            pl.BlockSpec((M, 1), lambda e: (e, 0)),
        ],
        out_specs=pl.BlockSpec((M, D), lambda e: (e, 0)),
        out_shape=jax.ShapeDtypeStruct((G, D), jnp.float32),
    )(xg, w_gate, w_up, w_down, gscale)


SB = 256
NSB = T // SB


def _shared_body(x_ref, y_ref, swg_ref, swu_ref, swd_ref, out_ref):
    xt = x_ref[...]
    a = jnp.dot(xt, swg_ref[...], preferred_element_type=jnp.float32)
    b = jnp.dot(xt, swu_ref[...], preferred_element_type=jnp.float32)
    h = a * lax.logistic(a) * b
    o = jnp.dot(h, swd_ref[...], preferred_element_type=jnp.float32)
    out_ref[...] = o + y_ref[0] + y_ref[1]


def _shared_combine(x, y2, sw_gate, sw_up, sw_down):
    return pl.pallas_call(
        _shared_body,
        grid=(NSB,),
        in_specs=[
            pl.BlockSpec((SB, D), lambda t: (t, 0)),
            pl.BlockSpec((2, SB, D), lambda t: (0, t, 0)),
            pl.BlockSpec((D, FF), lambda t: (0, 0)),
            pl.BlockSpec((D, FF), lambda t: (0, 0)),
            pl.BlockSpec((FF, D), lambda t: (0, 0)),
        ],
        out_specs=pl.BlockSpec((SB, D), lambda t: (t, 0)),
        out_shape=jax.ShapeDtypeStruct((T, D), jnp.float32),
    )(x, y2, sw_gate, sw_up, sw_down)


@jax.jit
def kernel(x, gate_w, w_gate, w_up, w_down, sw_gate, sw_up, sw_down):
    del gate_w
    gidx = jnp.asarray(_GIDX.reshape(NW, -1, 48))
    inv = jnp.asarray(_INV.reshape(NW, -1, 64))
    gscale = jnp.asarray(_GSCALE.reshape(G, 1))

    xg = _make_sc_gather(T, G, 48)(x, gidx)
    yg = _experts(xg, w_gate, w_up, w_down, gscale)
    y = _make_sc_gather(G, TOP_K * T, 64)(yg, inv)
    return _shared_combine(x, y.reshape(TOP_K, T, D), sw_gate, sw_up, sw_down)
